# baseline (speedup 1.0000x reference)
import jax
import jax.numpy as jnp
from jax import lax
from jax.experimental import pallas as pl
from jax.experimental.pallas import tpu as pltpu
from jax.experimental.pallas import tpu_sc as plsc

VOCAB = 1000000
DIM = 64
BATCH = 4096
HIST = 50

_INFO = plsc.get_sparse_core_info()
NC = _INFO.num_cores
NS = _INFO.num_subcores
NW = NC * NS
B_TOTAL = BATCH * HIST
B_PER_W = B_TOTAL // NW
CHUNK = 128
NG = B_PER_W // CHUNK
NBUF = 5

VB = 512
NBLK = VOCAB // VB
JBLK = NBLK // NW
TAIL = VOCAB - NBLK * VB
LANE_GROUPS = DIM // 16


def _xpose_body(pt_t, it_t, tail_pt, tail_it, rm_pt, rm_it, inb, outb,
                sem_i, sem_o):
    wid = lax.axis_index("s") * NC + lax.axis_index("c")

    iotas = [lax.iota(jnp.int32, 16) + 16 * q for q in range(LANE_GROUPS)]
    iotas_h = [it + DIM for it in iotas]

    def transpose_block(p, nrows):
        def row_body(o, c):
            base8 = jnp.full((16,), 8 * o, jnp.int32)
            for di in range(8):
                isplat = base8 + di
                rsplat = lax.shift_right_logical(isplat, 1)
                cols = iotas_h if di % 2 else iotas
                for q in range(LANE_GROUPS):
                    v = plsc.load_gather(inb.at[p], [iotas[q], isplat])
                    plsc.store_scatter(outb.at[p], [rsplat, cols[q]], v)
            return c

        lax.fori_loop(0, nrows // 8, row_body, 0)

    def one_table(src_t, dst_rm, nblk):
        def off(j):
            return pl.multiple_of(VB * (wid + NW * j), VB)

        def off2(j):
            return pl.multiple_of((VB // 2) * (wid + NW * j), VB // 2)

        def fire_in(j, p):
            pltpu.async_copy(src_t.at[:, pl.ds(off(j), VB)], inb.at[p],
                             sem_i[p])

        def wait_in(j, p):
            pltpu.make_async_copy(src_t.at[:, pl.ds(off(j), VB)], inb.at[p],
                                  sem_i[p]).wait()

        def fire_out(j, p):
            pltpu.async_copy(outb.at[p],
                             dst_rm.at[pl.ds(off2(j), VB // 2)],
                             sem_o[p])

        def wait_out(j, p):
            pltpu.make_async_copy(outb.at[p],
                                  dst_rm.at[pl.ds(off2(j), VB // 2)],
                                  sem_o[p]).wait()

        fire_in(0, 0)
        fire_in(1, 1)
        wait_in(0, 0)
        transpose_block(0, VB)
        fire_out(0, 0)
        fire_in(2, 0)
        wait_in(1, 1)
        transpose_block(1, VB)
        fire_out(1, 1)
        fire_in(3, 1)

        def body(o, c):
            for p in range(2):
                j = o * 2 + p
                wait_in(j, p)
                wait_out(j - 2, p)
                transpose_block(p, VB)
                fire_out(j, p)

                @pl.when(j + 2 < nblk)
                def _():
                    fire_in(j + 2, p)

            return c

        lax.fori_loop(1, nblk // 2, body, 0)
        if nblk % 2:
            j = nblk - 1
            wait_in(j, 0)
            wait_out(j - 2, 0)
            transpose_block(0, VB)
            fire_out(j, 0)
        wait_out(nblk - 2, (nblk - 2) % 2)
        wait_out(nblk - 1, (nblk - 1) % 2)

    one_table(pt_t, rm_pt, JBLK)
    one_table(it_t, rm_it, JBLK)

    @pl.when(wid == 0)
    def _():
        base = JBLK * NW * VB
        for src_t, dst_rm in ((pt_t, rm_pt), (it_t, rm_it)):
            pltpu.sync_copy(src_t.at[:, pl.ds(base, VB)], inb.at[0])
            transpose_block(0, VB)
            pltpu.sync_copy(outb.at[0], dst_rm.at[pl.ds(base // 2, VB // 2)])
        tbase = NBLK * VB // 2
        pltpu.sync_copy(tail_pt, rm_pt.at[pl.ds(tbase, TAIL // 2)])
        pltpu.sync_copy(tail_it, rm_it.at[pl.ds(tbase, TAIL // 2)])


def _emb_body(pt_hbm, it_hbm, idx_hbm, out_hbm, idx_v, va, vb, vst,
              sem_a, sem_b, sem_st):
    wid = lax.axis_index("s") * NC + lax.axis_index("c")
    out_base = wid * B_PER_W
    pltpu.sync_copy(idx_hbm.at[wid], idx_v)

    def fire_gathers(g, b):
        pltpu.async_copy(pt_hbm.at[idx_v.at[g]], va.at[b], sem_a[b])
        pltpu.async_copy(it_hbm.at[idx_v.at[g]], vb.at[b], sem_b[b])

    def wait_gathers(g, b):
        pltpu.make_async_copy(pt_hbm.at[idx_v.at[g]], va.at[b],
                              sem_a[b]).wait()
        pltpu.make_async_copy(it_hbm.at[idx_v.at[g]], vb.at[b],
                              sem_b[b]).wait()

    def out_slice(g):
        return out_hbm.at[pl.ds(out_base + g * CHUNK, CHUNK)]

    def add_and_store(g, b):
        def add_body(i, c):
            for j in range(LANE_GROUPS):
                s = pl.ds(j * 16, 16)
                vst[b, i, s] = va[b, i, s] + vb[b, i, s]
            return c

        lax.fori_loop(0, CHUNK, add_body, 0)
        pltpu.async_copy(vst.at[b], out_slice(g), sem_st[b])

    for b in range(NBUF):
        fire_gathers(b, b)

    for b in range(NBUF):
        wait_gathers(b, b)
        add_and_store(b, b)
        fire_gathers(b + NBUF, b)

    def outer_body(o, carry):
        for b in range(NBUF):
            g = o * NBUF + b
            wait_gathers(g, b)
            pltpu.make_async_copy(vst.at[b], out_slice(g - NBUF),
                                  sem_st[b]).wait()
            add_and_store(g, b)
            fire_gathers(g + NBUF, b)
        return carry

    lax.fori_loop(1, NG // NBUF - 1, outer_body, 0)

    for b in range(NBUF):
        g = NG - NBUF + b
        wait_gathers(g, b)
        pltpu.make_async_copy(vst.at[b], out_slice(g - NBUF),
                              sem_st[b]).wait()
        add_and_store(g, b)

    for b in range(NBUF):
        g = NG - NBUF + b
        pltpu.make_async_copy(vst.at[b], out_slice(g), sem_st[b]).wait()


@jax.jit
def _emb(pretrain_t, id_t, tail_pt, tail_it, idx):
    mesh = plsc.VectorSubcoreMesh(core_axis_name="c", subcore_axis_name="s")
    xpose = pl.kernel(
        _xpose_body,
        out_type=(
            jax.ShapeDtypeStruct((VOCAB // 2, 2 * DIM), jnp.float32),
            jax.ShapeDtypeStruct((VOCAB // 2, 2 * DIM), jnp.float32),
        ),
        mesh=mesh,
        scratch_types=[
            pltpu.VMEM((2, DIM, VB), jnp.float32),
            pltpu.VMEM((2, VB // 2, 2 * DIM), jnp.float32),
            [pltpu.SemaphoreType.DMA] * 2,
            [pltpu.SemaphoreType.DMA] * 2,
        ],
        compiler_params=pltpu.CompilerParams(
            use_tc_tiling_on_sc=True, needs_layout_passes=False),
    )
    rm_pt, rm_it = xpose(pretrain_t, id_t, tail_pt, tail_it)
    gather = pl.kernel(
        _emb_body,
        out_type=jax.ShapeDtypeStruct((B_TOTAL, DIM), jnp.float32),
        mesh=mesh,
        scratch_types=[
            pltpu.VMEM((NG, CHUNK), jnp.int32),
            pltpu.VMEM((NBUF, CHUNK, DIM), jnp.float32),
            pltpu.VMEM((NBUF, CHUNK, DIM), jnp.float32),
            pltpu.VMEM((NBUF, CHUNK, DIM), jnp.float32),
            [pltpu.SemaphoreType.DMA] * NBUF,
            [pltpu.SemaphoreType.DMA] * NBUF,
            [pltpu.SemaphoreType.DMA] * NBUF,
        ],
        compiler_params=pltpu.CompilerParams(
            use_tc_tiling_on_sc=False, needs_layout_passes=False),
    )
    return gather(rm_pt.reshape(VOCAB, DIM), rm_it.reshape(VOCAB, DIM), idx)


def kernel(inputs, pretrain_table, id_table):
    idx = inputs.reshape(NW, NG, CHUNK)
    tail_pt = pretrain_table[NBLK * VB:].reshape(TAIL // 2, 2 * DIM)
    tail_it = id_table[NBLK * VB:].reshape(TAIL // 2, 2 * DIM)
    out = _emb(pretrain_table.T, id_table.T, tail_pt, tail_it, idx)
    return out.reshape(BATCH, HIST, DIM)


# speedup vs baseline: 2.5261x; 2.5261x over previous
import jax
import jax.numpy as jnp
from jax import lax
from jax.experimental import pallas as pl
from jax.experimental.pallas import tpu as pltpu
from jax.experimental.pallas import tpu_sc as plsc

VOCAB = 1000000
DIM = 64
BATCH = 4096
HIST = 50

_INFO = plsc.get_sparse_core_info()
NC = _INFO.num_cores
NS = _INFO.num_subcores
NW = NC * NS
B_TOTAL = BATCH * HIST
B_PER_W = B_TOTAL // NW
CHUNK = 128
NG = B_PER_W // CHUNK
NBUF = 5
LANE_GROUPS = DIM // 16

def _emb_body(pt_hbm, it_hbm, idx_hbm, out_hbm, idx_v, va, vb, vst,
              sem_a, sem_b, sem_st):
    wid = lax.axis_index("s") * NC + lax.axis_index("c")
    out_base = wid * B_PER_W
    pltpu.sync_copy(idx_hbm.at[wid], idx_v)

    def fire_gathers(g, b):
        pltpu.async_copy(pt_hbm.at[idx_v.at[g]], va.at[b], sem_a[b])
        pltpu.async_copy(it_hbm.at[idx_v.at[g]], vb.at[b], sem_b[b])

    def wait_gathers(g, b):
        pltpu.make_async_copy(pt_hbm.at[idx_v.at[g]], va.at[b],
                              sem_a[b]).wait()
        pltpu.make_async_copy(it_hbm.at[idx_v.at[g]], vb.at[b],
                              sem_b[b]).wait()

    def out_slice(g):
        return out_hbm.at[pl.ds(out_base + g * CHUNK, CHUNK)]

    def add_and_store(g, b):
        def add_body(i, c):
            for j in range(LANE_GROUPS):
                s = pl.ds(j * 16, 16)
                vst[b, i, s] = va[b, i, s] + vb[b, i, s]
            return c

        lax.fori_loop(0, CHUNK, add_body, 0)
        pltpu.async_copy(vst.at[b], out_slice(g), sem_st[b])

    for b in range(NBUF):
        fire_gathers(b, b)

    for b in range(NBUF):
        wait_gathers(b, b)
        add_and_store(b, b)
        fire_gathers(b + NBUF, b)

    def outer_body(o, carry):
        for b in range(NBUF):
            g = o * NBUF + b
            wait_gathers(g, b)
            pltpu.make_async_copy(vst.at[b], out_slice(g - NBUF),
                                  sem_st[b]).wait()
            add_and_store(g, b)
            fire_gathers(g + NBUF, b)
        return carry

    lax.fori_loop(1, NG // NBUF - 1, outer_body, 0)

    for b in range(NBUF):
        g = NG - NBUF + b
        wait_gathers(g, b)
        pltpu.make_async_copy(vst.at[b], out_slice(g - NBUF),
                              sem_st[b]).wait()
        add_and_store(g, b)

    for b in range(NBUF):
        g = NG - NBUF + b
        pltpu.make_async_copy(vst.at[b], out_slice(g), sem_st[b]).wait()


@jax.jit
def _emb(pretrain_table, id_table, idx):
    rm_pt = pretrain_table
    rm_it = id_table
    mesh = plsc.VectorSubcoreMesh(core_axis_name="c", subcore_axis_name="s")
    gather = pl.kernel(
        _emb_body,
        out_type=jax.ShapeDtypeStruct((B_TOTAL, DIM), jnp.float32),
        mesh=mesh,
        scratch_types=[
            pltpu.VMEM((NG, CHUNK), jnp.int32),
            pltpu.VMEM((NBUF, CHUNK, DIM), jnp.float32),
            pltpu.VMEM((NBUF, CHUNK, DIM), jnp.float32),
            pltpu.VMEM((NBUF, CHUNK, DIM), jnp.float32),
            [pltpu.SemaphoreType.DMA] * NBUF,
            [pltpu.SemaphoreType.DMA] * NBUF,
            [pltpu.SemaphoreType.DMA] * NBUF,
        ],
        compiler_params=pltpu.CompilerParams(
            use_tc_tiling_on_sc=False, needs_layout_passes=False),
    )
    return gather(rm_pt, rm_it, idx)


def kernel(inputs, pretrain_table, id_table):
    idx = inputs.reshape(NW, NG, CHUNK)
    out = _emb(pretrain_table, id_table, idx)
    return out.reshape(BATCH, HIST, DIM)


# speedup vs baseline: 2.5319x; 1.0023x over previous
import jax
import jax.numpy as jnp
from jax import lax
from jax.experimental import pallas as pl
from jax.experimental.pallas import tpu as pltpu
from jax.experimental.pallas import tpu_sc as plsc

VOCAB = 1000000
DIM = 64
BATCH = 4096
HIST = 50

_INFO = plsc.get_sparse_core_info()
NC = _INFO.num_cores
NS = _INFO.num_subcores
NW = NC * NS
B_TOTAL = BATCH * HIST
B_PER_W = B_TOTAL // NW
CHUNK = 128
NG = B_PER_W // CHUNK
NBUF = 5
LANE_GROUPS = DIM // 16

def _emb_body(pt_hbm, it_hbm, idx_hbm, out_hbm, idx_v, va, vb, vst,
              sem_a, sem_b, sem_st):
    wid = lax.axis_index("s") * NC + lax.axis_index("c")
    out_base = wid * B_PER_W
    pltpu.sync_copy(idx_hbm.at[wid], idx_v)

    def fire_gathers(g, b):
        pltpu.async_copy(pt_hbm.at[idx_v.at[g]], va.at[b], sem_a[b])
        pltpu.async_copy(it_hbm.at[idx_v.at[g]], vb.at[b], sem_b[b])

    def wait_gathers(g, b):
        pltpu.make_async_copy(pt_hbm.at[idx_v.at[g]], va.at[b],
                              sem_a[b]).wait()
        pltpu.make_async_copy(it_hbm.at[idx_v.at[g]], vb.at[b],
                              sem_b[b]).wait()

    def out_slice(g):
        return out_hbm.at[pl.ds(out_base + g * CHUNK, CHUNK)]

    def add_and_store(g, b):
        def add_body(i, c):
            for j in range(LANE_GROUPS):
                s = pl.ds(j * 16, 16)
                vst[b, i, s] = va[b, i, s] + vb[b, i, s]
            return c

        lax.fori_loop(0, CHUNK, add_body, 0)
        pltpu.async_copy(vst.at[b], out_slice(g), sem_st[b])

    for b in range(NBUF):
        fire_gathers(b, b)

    for b in range(NBUF):
        wait_gathers(b, b)
        add_and_store(b, b)
        fire_gathers(b + NBUF, b)

    def outer_body(o, carry):
        for b in range(NBUF):
            g = o * NBUF + b
            wait_gathers(g, b)
            pltpu.make_async_copy(vst.at[b], out_slice(g - NBUF),
                                  sem_st[b]).wait()
            add_and_store(g, b)
            fire_gathers(g + NBUF, b)
        return carry

    lax.fori_loop(1, NG // NBUF - 1, outer_body, 0)

    for b in range(NBUF):
        g = NG - NBUF + b
        wait_gathers(g, b)
        pltpu.make_async_copy(vst.at[b], out_slice(g - NBUF),
                              sem_st[b]).wait()
        add_and_store(g, b)

    for b in range(NBUF):
        g = NG - NBUF + b
        pltpu.make_async_copy(vst.at[b], out_slice(g), sem_st[b]).wait()


@jax.jit
def _emb(pretrain_table, id_table, idx):
    mesh = plsc.VectorSubcoreMesh(core_axis_name="c", subcore_axis_name="s")
    gather = pl.kernel(
        _emb_body,
        out_type=jax.ShapeDtypeStruct((B_TOTAL, DIM), jnp.float32),
        mesh=mesh,
        scratch_types=[
            pltpu.VMEM((NG, CHUNK), jnp.int32),
            pltpu.VMEM((NBUF, CHUNK, DIM), jnp.float32),
            pltpu.VMEM((NBUF, CHUNK, DIM), jnp.float32),
            pltpu.VMEM((NBUF, CHUNK, DIM), jnp.float32),
            [pltpu.SemaphoreType.DMA] * NBUF,
            [pltpu.SemaphoreType.DMA] * NBUF,
            [pltpu.SemaphoreType.DMA] * NBUF,
        ],
        compiler_params=pltpu.CompilerParams(
            use_tc_tiling_on_sc=False, needs_layout_passes=False),
    )
    return gather(pretrain_table, id_table, idx)


def kernel(inputs, pretrain_table, id_table):
    idx = inputs.reshape(NW, NG, CHUNK)
    out = _emb(pretrain_table, id_table, idx)
    return out.reshape(BATCH, HIST, DIM)
